# Initial kernel scaffold; baseline (speedup 1.0000x reference)
import jax
import jax.numpy as jnp
from jax.experimental import pallas as pl


def kernel(atom_features, distance, atom_membership, distance_membership_i, distance_membership_j, W_cf, W_df, W_fc, b_cf, b_df):
    raise NotImplementedError("write your pallas kernel here")


# speedup vs baseline: 3.1800x; 3.1800x over previous
import functools

import jax
import jax.numpy as jnp
from jax import lax
from jax.experimental import pallas as pl
from jax.experimental.pallas import tpu as pltpu
from jax.experimental.pallas import tpu_sc as plsc

N_ATOMS = 10000
N_PAIRS = 320000
N_EMB = 128
N_DIST = 100
N_HID = 64

NC = 2
NS = 16
NW = NC * NS

_B1 = 1000


def _tc1_body(af_ref, wcf_ref, wfc_ref, bcf_ref, bdf_ref, afh_ref, hbase_ref):
    afh = jnp.dot(af_ref[...], wcf_ref[...],
                  preferred_element_type=jnp.float32) + bcf_ref[...]
    afh_ref[...] = jnp.concatenate([afh, jnp.zeros_like(afh)], axis=1)
    oii = jnp.tanh(jnp.dot(afh * bdf_ref[...], wfc_ref[...],
                           preferred_element_type=jnp.float32))
    hbase_ref[...] = 0.5 * (af_ref[...] - oii)


def _tc1(af, wcf, wfc, bcf, bdf):
    grid = (N_ATOMS // _B1,)
    return pl.pallas_call(
        _tc1_body,
        grid=grid,
        in_specs=[
            pl.BlockSpec((_B1, N_EMB), lambda i: (i, 0)),
            pl.BlockSpec((N_EMB, N_HID), lambda i: (0, 0)),
            pl.BlockSpec((N_HID, N_EMB), lambda i: (0, 0)),
            pl.BlockSpec((1, N_HID), lambda i: (0, 0)),
            pl.BlockSpec((1, N_HID), lambda i: (0, 0)),
        ],
        out_specs=[
            pl.BlockSpec((_B1, N_EMB), lambda i: (i, 0)),
            pl.BlockSpec((_B1, N_EMB), lambda i: (i, 0)),
        ],
        out_shape=[
            jax.ShapeDtypeStruct((N_ATOMS, N_EMB), jnp.float32),
            jax.ShapeDtypeStruct((N_ATOMS, N_EMB), jnp.float32),
        ],
    )(af, wcf, wfc, bcf, bdf)


_GB = N_PAIRS // NW
_GCH = 1000


def _sc_gather_body(afh_hbm, idx_hbm, out_hbm, idx_v, rows_v, sem):
    wid = lax.axis_index("s") * NC + lax.axis_index("c")
    base = wid * _GB

    def body(i, carry):
        off = pl.multiple_of(base + i * _GCH, 8)
        pltpu.sync_copy(idx_hbm.at[pl.ds(off, _GCH)], idx_v)
        pltpu.async_copy(afh_hbm.at[idx_v], rows_v, sem).wait()
        pltpu.sync_copy(rows_v, out_hbm.at[pl.ds(off, _GCH)])
        return carry

    lax.fori_loop(0, _GB // _GCH, body, 0)


def _sc_gather(afh, dmj):
    mesh = plsc.VectorSubcoreMesh(core_axis_name="c", subcore_axis_name="s",
                                  num_cores=NC, num_subcores=NS)
    call = pl.kernel(
        _sc_gather_body,
        out_type=jax.ShapeDtypeStruct((N_PAIRS, N_EMB), jnp.float32),
        mesh=mesh,
        scratch_types=[
            pltpu.VMEM((_GCH,), jnp.int32),
            pltpu.VMEM((_GCH, N_EMB), jnp.float32),
            pltpu.SemaphoreType.DMA,
        ],
    )
    return call(afh, dmj)


_B2 = 3200


def _tc2_body(d_ref, g_ref, wdf_ref, wfc_ref, bdf_ref, y_ref):
    dh = jnp.dot(d_ref[...], wdf_ref[...],
                 preferred_element_type=jnp.float32) + bdf_ref[...]
    g = g_ref[:, :N_HID]
    y_ref[...] = jnp.tanh(jnp.dot(dh * g, wfc_ref[...],
                                  preferred_element_type=jnp.float32))


def _tc2(dist, gathered, wdf, wfc, bdf):
    grid = (N_PAIRS // _B2,)
    return pl.pallas_call(
        _tc2_body,
        grid=grid,
        in_specs=[
            pl.BlockSpec((_B2, N_DIST), lambda i: (i, 0)),
            pl.BlockSpec((_B2, N_EMB), lambda i: (i, 0)),
            pl.BlockSpec((N_DIST, N_HID), lambda i: (0, 0)),
            pl.BlockSpec((N_HID, N_EMB), lambda i: (0, 0)),
            pl.BlockSpec((1, N_HID), lambda i: (0, 0)),
        ],
        out_specs=pl.BlockSpec((_B2, N_EMB), lambda i: (i, 0)),
        out_shape=jax.ShapeDtypeStruct((N_PAIRS, N_EMB), jnp.float32),
    )(dist, gathered, wdf, wfc, bdf)


_P_SC = N_PAIRS // NC
_P_TILE = _P_SC // NS
_SCH = 200
_ROWS_T = 624
_ROWS_TAIL = N_ATOMS - NS * _ROWS_T


def _sc_scatter_body(y_hbm, dmi_hbm, hbase_hbm, out_hbm, idx_v, rows_v,
                     acc_sh, sem):
    c = lax.axis_index("c")
    s = lax.axis_index("s")
    pltpu.sync_copy(hbase_hbm.at[pl.ds(s * _ROWS_T, _ROWS_T)],
                    acc_sh.at[pl.ds(s * _ROWS_T, _ROWS_T)])

    @pl.when(s == 0)
    def _():
        pltpu.sync_copy(hbase_hbm.at[pl.ds(NS * _ROWS_T, _ROWS_TAIL)],
                        acc_sh.at[pl.ds(NS * _ROWS_T, _ROWS_TAIL)])

    plsc.subcore_barrier()

    base = c * _P_SC + s * _P_TILE

    def body(i, carry):
        off = pl.multiple_of(base + i * _SCH, 8)
        pltpu.sync_copy(dmi_hbm.at[pl.ds(off, _SCH)], idx_v)
        pltpu.sync_copy(y_hbm.at[pl.ds(off, _SCH)], rows_v)
        pltpu.sync_copy(rows_v, acc_sh.at[idx_v], add=True)
        return carry

    lax.fori_loop(0, _P_TILE // _SCH, body, 0)
    plsc.subcore_barrier()
    pltpu.sync_copy(acc_sh.at[pl.ds(s * _ROWS_T, _ROWS_T)],
                    out_hbm.at[c, pl.ds(s * _ROWS_T, _ROWS_T)])

    @pl.when(s == 0)
    def _():
        pltpu.sync_copy(acc_sh.at[pl.ds(NS * _ROWS_T, _ROWS_TAIL)],
                        out_hbm.at[c, pl.ds(NS * _ROWS_T, _ROWS_TAIL)])


def _sc_scatter(y, dmi, hbase):
    mesh = plsc.VectorSubcoreMesh(core_axis_name="c", subcore_axis_name="s",
                                  num_cores=NC, num_subcores=NS)
    call = pl.kernel(
        _sc_scatter_body,
        out_type=jax.ShapeDtypeStruct((NC, N_ATOMS, N_EMB), jnp.float32),
        mesh=mesh,
        scratch_types=[
            pltpu.VMEM((_SCH,), jnp.int32),
            pltpu.VMEM((_SCH, N_EMB), jnp.float32),
            pltpu.VMEM_SHARED((N_ATOMS, N_EMB), jnp.float32),
            pltpu.SemaphoreType.DMA,
        ],
    )
    return call(y, dmi, hbase)


def _tc3_body(acc_ref, out_ref):
    out_ref[...] = acc_ref[0] + acc_ref[1]


def _tc3(acc):
    grid = (N_ATOMS // _B1,)
    return pl.pallas_call(
        _tc3_body,
        grid=grid,
        in_specs=[pl.BlockSpec((NC, _B1, N_EMB), lambda i: (0, i, 0))],
        out_specs=pl.BlockSpec((_B1, N_EMB), lambda i: (i, 0)),
        out_shape=jax.ShapeDtypeStruct((N_ATOMS, N_EMB), jnp.float32),
    )(acc)


def kernel(atom_features, distance, atom_membership, distance_membership_i,
           distance_membership_j, W_cf, W_df, W_fc, b_cf, b_df):
    del atom_membership
    dmi = distance_membership_i.astype(jnp.int32)
    dmj = distance_membership_j.astype(jnp.int32)
    bcf2 = b_cf.reshape(1, N_HID)
    bdf2 = b_df.reshape(1, N_HID)

    afh, hbase = _tc1(atom_features, W_cf, W_fc, bcf2, bdf2)
    gathered = _sc_gather(afh, dmj)
    y = _tc2(distance, gathered, W_df, W_fc, bdf2)
    acc = _sc_scatter(y, dmi, hbase)
    return _tc3(acc)


# speedup vs baseline: 3.5489x; 1.1160x over previous
import functools

import jax
import jax.numpy as jnp
from jax import lax
from jax.experimental import pallas as pl
from jax.experimental.pallas import tpu as pltpu
from jax.experimental.pallas import tpu_sc as plsc

N_ATOMS = 10000
N_PAIRS = 320000
N_EMB = 128
N_DIST = 100
N_HID = 64

NC = 2
NS = 16
NW = NC * NS

_B1 = 1000


def _tc1_body(af_ref, wcf_ref, wfc_ref, bcf_ref, bdf_ref, afh_ref, hbase_ref):
    afh = jnp.dot(af_ref[...], wcf_ref[...],
                  preferred_element_type=jnp.float32) + bcf_ref[...]
    afh_ref[...] = jnp.concatenate([afh, jnp.zeros_like(afh)], axis=1)
    oii = jnp.tanh(jnp.dot(afh * bdf_ref[...], wfc_ref[...],
                           preferred_element_type=jnp.float32))
    hbase_ref[...] = 0.5 * (af_ref[...] - oii)


def _tc1(af, wcf, wfc, bcf, bdf):
    grid = (N_ATOMS // _B1,)
    return pl.pallas_call(
        _tc1_body,
        grid=grid,
        in_specs=[
            pl.BlockSpec((_B1, N_EMB), lambda i: (i, 0)),
            pl.BlockSpec((N_EMB, N_HID), lambda i: (0, 0)),
            pl.BlockSpec((N_HID, N_EMB), lambda i: (0, 0)),
            pl.BlockSpec((1, N_HID), lambda i: (0, 0)),
            pl.BlockSpec((1, N_HID), lambda i: (0, 0)),
        ],
        out_specs=[
            pl.BlockSpec((_B1, N_EMB), lambda i: (i, 0)),
            pl.BlockSpec((_B1, N_EMB), lambda i: (i, 0)),
        ],
        out_shape=[
            jax.ShapeDtypeStruct((N_ATOMS, N_EMB), jnp.float32),
            jax.ShapeDtypeStruct((N_ATOMS, N_EMB), jnp.float32),
        ],
    )(af, wcf, wfc, bcf, bdf)


_GB = N_PAIRS // NW
_GCH = 400
_GN = _GB // _GCH


def _sc_gather_body(afh_hbm, idx_hbm, out_hbm, idx_v, rows_v, sem_g, sem_s):
    wid = lax.axis_index("s") * NC + lax.axis_index("c")
    base = wid * _GB
    pltpu.sync_copy(idx_hbm.at[pl.ds(base, _GB)], idx_v)

    def issue_gather(i, b):
        off = pl.multiple_of(i * _GCH, 8)
        pltpu.async_copy(afh_hbm.at[idx_v.at[pl.ds(off, _GCH)]],
                         rows_v.at[b], sem_g.at[b])

    def drain(sem, b):
        pltpu.make_async_copy(afh_hbm.at[pl.ds(0, _GCH)], rows_v.at[b],
                              sem.at[b]).wait()

    issue_gather(0, 0)

    def body(i, carry):
        b = i % 2
        nb = 1 - b

        @pl.when(i + 1 < _GN)
        def _():
            @pl.when(i >= 1)
            def _():
                drain(sem_s, nb)
            issue_gather(i + 1, nb)

        drain(sem_g, b)
        off = pl.multiple_of(base + i * _GCH, 8)
        pltpu.async_copy(rows_v.at[b], out_hbm.at[pl.ds(off, _GCH)],
                         sem_s.at[b])
        return carry

    lax.fori_loop(0, _GN, body, 0)
    drain(sem_s, 0)
    drain(sem_s, 1)


def _sc_gather(afh, dmj):
    mesh = plsc.VectorSubcoreMesh(core_axis_name="c", subcore_axis_name="s",
                                  num_cores=NC, num_subcores=NS)
    call = pl.kernel(
        _sc_gather_body,
        out_type=jax.ShapeDtypeStruct((N_PAIRS, N_EMB), jnp.float32),
        mesh=mesh,
        scratch_types=[
            pltpu.VMEM((_GB,), jnp.int32),
            pltpu.VMEM((2, _GCH, N_EMB), jnp.float32),
            pltpu.SemaphoreType.DMA((2,)),
            pltpu.SemaphoreType.DMA((2,)),
        ],
    )
    return call(afh, dmj)


_B2 = 3200


def _tc2_body(d_ref, g_ref, wdf_ref, wfc_ref, bdf_ref, y_ref):
    dh = jnp.dot(d_ref[...], wdf_ref[...],
                 preferred_element_type=jnp.float32) + bdf_ref[...]
    g = g_ref[:, :N_HID]
    y_ref[...] = jnp.tanh(jnp.dot(dh * g, wfc_ref[...],
                                  preferred_element_type=jnp.float32))


def _tc2(dist, gathered, wdf, wfc, bdf):
    grid = (N_PAIRS // _B2,)
    return pl.pallas_call(
        _tc2_body,
        grid=grid,
        in_specs=[
            pl.BlockSpec((_B2, N_DIST), lambda i: (i, 0)),
            pl.BlockSpec((_B2, N_EMB), lambda i: (i, 0)),
            pl.BlockSpec((N_DIST, N_HID), lambda i: (0, 0)),
            pl.BlockSpec((N_HID, N_EMB), lambda i: (0, 0)),
            pl.BlockSpec((1, N_HID), lambda i: (0, 0)),
        ],
        out_specs=pl.BlockSpec((_B2, N_EMB), lambda i: (i, 0)),
        out_shape=jax.ShapeDtypeStruct((N_PAIRS, N_EMB), jnp.float32),
    )(dist, gathered, wdf, wfc, bdf)


_P_SC = N_PAIRS // NC
_SCH = 128
_P_T0 = 10112
_N_T0 = _P_T0 // _SCH
_N_T15 = (_P_SC - (NS - 1) * _P_T0) // _SCH
_ROWS_T = 624
_ROWS_TAIL = N_ATOMS - NS * _ROWS_T


def _sc_scatter_body(y_hbm, dmi_hbm, hbase_hbm, out_hbm, idx_v, rows_v,
                     acc_sh, sem_i, sem_g, sem_a):
    c = lax.axis_index("c")
    s = lax.axis_index("s")
    pltpu.sync_copy(hbase_hbm.at[pl.ds(s * _ROWS_T, _ROWS_T)],
                    acc_sh.at[pl.ds(s * _ROWS_T, _ROWS_T)])

    @pl.when(s == 0)
    def _():
        pltpu.sync_copy(hbase_hbm.at[pl.ds(NS * _ROWS_T, _ROWS_TAIL)],
                        acc_sh.at[pl.ds(NS * _ROWS_T, _ROWS_TAIL)])

    plsc.subcore_barrier()

    base = c * _P_SC + s * _P_T0
    n = jnp.where(s == NS - 1, _N_T15, _N_T0)

    def issue_load(i, b):
        off = pl.multiple_of(base + i * _SCH, 8)
        pltpu.async_copy(dmi_hbm.at[pl.ds(off, _SCH)], idx_v.at[b],
                         sem_i.at[b])
        pltpu.async_copy(y_hbm.at[pl.ds(off, _SCH)], rows_v.at[b],
                         sem_g.at[b])

    def drain_i(b):
        pltpu.make_async_copy(dmi_hbm.at[pl.ds(0, _SCH)], idx_v.at[b],
                              sem_i.at[b]).wait()

    def drain_rows(sem, b):
        pltpu.make_async_copy(y_hbm.at[pl.ds(0, _SCH)], rows_v.at[b],
                              sem.at[b]).wait()

    issue_load(0, 0)

    def body(i, carry):
        b = i % 2
        nb = 1 - b

        @pl.when(i + 1 < n)
        def _():
            @pl.when(i >= 1)
            def _():
                drain_rows(sem_a, nb)
            issue_load(i + 1, nb)

        drain_i(b)
        drain_rows(sem_g, b)
        pltpu.async_copy(rows_v.at[b], acc_sh.at[idx_v.at[b]], sem_a.at[b],
                         add=True)
        return carry

    lax.fori_loop(0, n, body, 0)
    drain_rows(sem_a, 0)
    drain_rows(sem_a, 1)
    plsc.subcore_barrier()
    pltpu.sync_copy(acc_sh.at[pl.ds(s * _ROWS_T, _ROWS_T)],
                    out_hbm.at[c, pl.ds(s * _ROWS_T, _ROWS_T)])

    @pl.when(s == 0)
    def _():
        pltpu.sync_copy(acc_sh.at[pl.ds(NS * _ROWS_T, _ROWS_TAIL)],
                        out_hbm.at[c, pl.ds(NS * _ROWS_T, _ROWS_TAIL)])


def _sc_scatter(y, dmi, hbase):
    mesh = plsc.VectorSubcoreMesh(core_axis_name="c", subcore_axis_name="s",
                                  num_cores=NC, num_subcores=NS)
    call = pl.kernel(
        _sc_scatter_body,
        out_type=jax.ShapeDtypeStruct((NC, N_ATOMS, N_EMB), jnp.float32),
        mesh=mesh,
        scratch_types=[
            pltpu.VMEM((2, _SCH), jnp.int32),
            pltpu.VMEM((2, _SCH, N_EMB), jnp.float32),
            pltpu.VMEM_SHARED((N_ATOMS, N_EMB), jnp.float32),
            pltpu.SemaphoreType.DMA((2,)),
            pltpu.SemaphoreType.DMA((2,)),
            pltpu.SemaphoreType.DMA((2,)),
        ],
    )
    return call(y, dmi, hbase)


def _tc3_body(acc_ref, out_ref):
    out_ref[...] = acc_ref[0] + acc_ref[1]


def _tc3(acc):
    grid = (N_ATOMS // _B1,)
    return pl.pallas_call(
        _tc3_body,
        grid=grid,
        in_specs=[pl.BlockSpec((NC, _B1, N_EMB), lambda i: (0, i, 0))],
        out_specs=pl.BlockSpec((_B1, N_EMB), lambda i: (i, 0)),
        out_shape=jax.ShapeDtypeStruct((N_ATOMS, N_EMB), jnp.float32),
    )(acc)


def kernel(atom_features, distance, atom_membership, distance_membership_i,
           distance_membership_j, W_cf, W_df, W_fc, b_cf, b_df):
    del atom_membership
    dmi = distance_membership_i.astype(jnp.int32)
    dmj = distance_membership_j.astype(jnp.int32)
    bcf2 = b_cf.reshape(1, N_HID)
    bdf2 = b_df.reshape(1, N_HID)

    afh, hbase = _tc1(atom_features, W_cf, W_fc, bcf2, bdf2)
    gathered = _sc_gather(afh, dmj)
    y = _tc2(distance, gathered, W_df, W_fc, bdf2)
    acc = _sc_scatter(y, dmi, hbase)
    return _tc3(acc)


# speedup vs baseline: 3.8915x; 1.0965x over previous
import functools

import jax
import jax.numpy as jnp
from jax import lax
from jax.experimental import pallas as pl
from jax.experimental.pallas import tpu as pltpu
from jax.experimental.pallas import tpu_sc as plsc

N_ATOMS = 10000
N_PAIRS = 320000
N_EMB = 128
N_DIST = 100
N_HID = 64

NC = 2
NS = 16
NW = NC * NS

_B1 = 1000


def _tc1_body(af_ref, wcf_ref, wfc_ref, bcf_ref, bdf_ref, afh_ref, hbase_ref):
    afh = jnp.dot(af_ref[...], wcf_ref[...],
                  preferred_element_type=jnp.float32) + bcf_ref[...]
    afh_ref[...] = jnp.concatenate([afh, jnp.zeros_like(afh)], axis=1)
    oii = jnp.tanh(jnp.dot(afh * bdf_ref[...], wfc_ref[...],
                           preferred_element_type=jnp.float32))
    hbase_ref[...] = 0.5 * (af_ref[...] - oii)


def _tc1(af, wcf, wfc, bcf, bdf):
    grid = (N_ATOMS // _B1,)
    return pl.pallas_call(
        _tc1_body,
        grid=grid,
        in_specs=[
            pl.BlockSpec((_B1, N_EMB), lambda i: (i, 0)),
            pl.BlockSpec((N_EMB, N_HID), lambda i: (0, 0)),
            pl.BlockSpec((N_HID, N_EMB), lambda i: (0, 0)),
            pl.BlockSpec((1, N_HID), lambda i: (0, 0)),
            pl.BlockSpec((1, N_HID), lambda i: (0, 0)),
        ],
        out_specs=[
            pl.BlockSpec((_B1, N_EMB), lambda i: (i, 0)),
            pl.BlockSpec((_B1, N_EMB), lambda i: (i, 0)),
        ],
        out_shape=[
            jax.ShapeDtypeStruct((N_ATOMS, N_EMB), jnp.float32),
            jax.ShapeDtypeStruct((N_ATOMS, N_EMB), jnp.float32),
        ],
    )(af, wcf, wfc, bcf, bdf)


_P_SC = N_PAIRS // NC
_SCH = 128
_P_T0 = 10112
_N_T0 = _P_T0 // _SCH
_N_T15 = (_P_SC - (NS - 1) * _P_T0) // _SCH
_ROWS_T = 624
_ROWS_TAIL = N_ATOMS - NS * _ROWS_T


def _sc_gather_body(afh_hbm, idx_hbm, out_hbm, idx_v, rows_v, afh_sh,
                    sem_i, sem_g, sem_s):
    c = lax.axis_index("c")
    s = lax.axis_index("s")
    pltpu.sync_copy(afh_hbm.at[pl.ds(s * _ROWS_T, _ROWS_T)],
                    afh_sh.at[pl.ds(s * _ROWS_T, _ROWS_T)])

    @pl.when(s == 0)
    def _():
        pltpu.sync_copy(afh_hbm.at[pl.ds(NS * _ROWS_T, _ROWS_TAIL)],
                        afh_sh.at[pl.ds(NS * _ROWS_T, _ROWS_TAIL)])

    plsc.subcore_barrier()

    base = c * _P_SC + s * _P_T0
    n = jnp.where(s == NS - 1, _N_T15, _N_T0)

    def issue_idx(i, b):
        off = pl.multiple_of(base + i * _SCH, 8)
        pltpu.async_copy(idx_hbm.at[pl.ds(off, _SCH)], idx_v.at[b],
                         sem_i.at[b])

    def drain_i(b):
        pltpu.make_async_copy(idx_hbm.at[pl.ds(0, _SCH)], idx_v.at[b],
                              sem_i.at[b]).wait()

    def drain_rows(sem, b):
        pltpu.make_async_copy(out_hbm.at[pl.ds(0, _SCH)], rows_v.at[b],
                              sem.at[b]).wait()

    issue_idx(0, 0)

    def body(i, carry):
        b = i % 2
        nb = 1 - b

        @pl.when(i + 1 < n)
        def _():
            issue_idx(i + 1, nb)

        @pl.when(i >= 2)
        def _():
            drain_rows(sem_s, b)

        drain_i(b)
        pltpu.async_copy(afh_sh.at[idx_v.at[b]], rows_v.at[b], sem_g.at[b])
        drain_rows(sem_g, b)
        off = pl.multiple_of(base + i * _SCH, 8)
        pltpu.async_copy(rows_v.at[b], out_hbm.at[pl.ds(off, _SCH)],
                         sem_s.at[b])
        return carry

    lax.fori_loop(0, n, body, 0)
    drain_rows(sem_s, 0)
    drain_rows(sem_s, 1)


def _sc_gather(afh, dmj):
    mesh = plsc.VectorSubcoreMesh(core_axis_name="c", subcore_axis_name="s",
                                  num_cores=NC, num_subcores=NS)
    call = pl.kernel(
        _sc_gather_body,
        out_type=jax.ShapeDtypeStruct((N_PAIRS, N_EMB), jnp.float32),
        mesh=mesh,
        scratch_types=[
            pltpu.VMEM((2, _SCH), jnp.int32),
            pltpu.VMEM((2, _SCH, N_EMB), jnp.float32),
            pltpu.VMEM_SHARED((N_ATOMS, N_EMB), jnp.float32),
            pltpu.SemaphoreType.DMA((2,)),
            pltpu.SemaphoreType.DMA((2,)),
            pltpu.SemaphoreType.DMA((2,)),
        ],
    )
    return call(afh, dmj)


_B2 = 3200


def _tc2_body(d_ref, g_ref, wdf_ref, wfc_ref, bdf_ref, y_ref):
    dh = jnp.dot(d_ref[...], wdf_ref[...],
                 preferred_element_type=jnp.float32) + bdf_ref[...]
    g = g_ref[:, :N_HID]
    y_ref[...] = jnp.tanh(jnp.dot(dh * g, wfc_ref[...],
                                  preferred_element_type=jnp.float32))


def _tc2(dist, gathered, wdf, wfc, bdf):
    grid = (N_PAIRS // _B2,)
    return pl.pallas_call(
        _tc2_body,
        grid=grid,
        in_specs=[
            pl.BlockSpec((_B2, N_DIST), lambda i: (i, 0)),
            pl.BlockSpec((_B2, N_EMB), lambda i: (i, 0)),
            pl.BlockSpec((N_DIST, N_HID), lambda i: (0, 0)),
            pl.BlockSpec((N_HID, N_EMB), lambda i: (0, 0)),
            pl.BlockSpec((1, N_HID), lambda i: (0, 0)),
        ],
        out_specs=pl.BlockSpec((_B2, N_EMB), lambda i: (i, 0)),
        out_shape=jax.ShapeDtypeStruct((N_PAIRS, N_EMB), jnp.float32),
    )(dist, gathered, wdf, wfc, bdf)


def _sc_scatter_body(y_hbm, dmi_hbm, hbase_hbm, out_hbm, idx_v, rows_v,
                     acc_sh, sem_i, sem_g, sem_a):
    c = lax.axis_index("c")
    s = lax.axis_index("s")
    pltpu.sync_copy(hbase_hbm.at[pl.ds(s * _ROWS_T, _ROWS_T)],
                    acc_sh.at[pl.ds(s * _ROWS_T, _ROWS_T)])

    @pl.when(s == 0)
    def _():
        pltpu.sync_copy(hbase_hbm.at[pl.ds(NS * _ROWS_T, _ROWS_TAIL)],
                        acc_sh.at[pl.ds(NS * _ROWS_T, _ROWS_TAIL)])

    plsc.subcore_barrier()

    base = c * _P_SC + s * _P_T0
    n = jnp.where(s == NS - 1, _N_T15, _N_T0)

    def issue_load(i, b):
        off = pl.multiple_of(base + i * _SCH, 8)
        pltpu.async_copy(dmi_hbm.at[pl.ds(off, _SCH)], idx_v.at[b],
                         sem_i.at[b])
        pltpu.async_copy(y_hbm.at[pl.ds(off, _SCH)], rows_v.at[b],
                         sem_g.at[b])

    def drain_i(b):
        pltpu.make_async_copy(dmi_hbm.at[pl.ds(0, _SCH)], idx_v.at[b],
                              sem_i.at[b]).wait()

    def drain_rows(sem, b):
        pltpu.make_async_copy(y_hbm.at[pl.ds(0, _SCH)], rows_v.at[b],
                              sem.at[b]).wait()

    issue_load(0, 0)

    def body(i, carry):
        b = i % 2
        nb = 1 - b

        @pl.when(i + 1 < n)
        def _():
            @pl.when(i >= 1)
            def _():
                drain_rows(sem_a, nb)
            issue_load(i + 1, nb)

        drain_i(b)
        drain_rows(sem_g, b)
        pltpu.async_copy(rows_v.at[b], acc_sh.at[idx_v.at[b]], sem_a.at[b],
                         add=True)
        return carry

    lax.fori_loop(0, n, body, 0)
    drain_rows(sem_a, 0)
    drain_rows(sem_a, 1)
    plsc.subcore_barrier()
    pltpu.sync_copy(acc_sh.at[pl.ds(s * _ROWS_T, _ROWS_T)],
                    out_hbm.at[c, pl.ds(s * _ROWS_T, _ROWS_T)])

    @pl.when(s == 0)
    def _():
        pltpu.sync_copy(acc_sh.at[pl.ds(NS * _ROWS_T, _ROWS_TAIL)],
                        out_hbm.at[c, pl.ds(NS * _ROWS_T, _ROWS_TAIL)])


def _sc_scatter(y, dmi, hbase):
    mesh = plsc.VectorSubcoreMesh(core_axis_name="c", subcore_axis_name="s",
                                  num_cores=NC, num_subcores=NS)
    call = pl.kernel(
        _sc_scatter_body,
        out_type=jax.ShapeDtypeStruct((NC, N_ATOMS, N_EMB), jnp.float32),
        mesh=mesh,
        scratch_types=[
            pltpu.VMEM((2, _SCH), jnp.int32),
            pltpu.VMEM((2, _SCH, N_EMB), jnp.float32),
            pltpu.VMEM_SHARED((N_ATOMS, N_EMB), jnp.float32),
            pltpu.SemaphoreType.DMA((2,)),
            pltpu.SemaphoreType.DMA((2,)),
            pltpu.SemaphoreType.DMA((2,)),
        ],
    )
    return call(y, dmi, hbase)


def _tc3_body(acc_ref, out_ref):
    out_ref[...] = acc_ref[0] + acc_ref[1]


def _tc3(acc):
    grid = (N_ATOMS // _B1,)
    return pl.pallas_call(
        _tc3_body,
        grid=grid,
        in_specs=[pl.BlockSpec((NC, _B1, N_EMB), lambda i: (0, i, 0))],
        out_specs=pl.BlockSpec((_B1, N_EMB), lambda i: (i, 0)),
        out_shape=jax.ShapeDtypeStruct((N_ATOMS, N_EMB), jnp.float32),
    )(acc)


def kernel(atom_features, distance, atom_membership, distance_membership_i,
           distance_membership_j, W_cf, W_df, W_fc, b_cf, b_df):
    del atom_membership
    dmi = distance_membership_i.astype(jnp.int32)
    dmj = distance_membership_j.astype(jnp.int32)
    bcf2 = b_cf.reshape(1, N_HID)
    bdf2 = b_df.reshape(1, N_HID)

    afh, hbase = _tc1(atom_features, W_cf, W_fc, bcf2, bdf2)
    gathered = _sc_gather(afh, dmj)
    y = _tc2(distance, gathered, W_df, W_fc, bdf2)
    acc = _sc_scatter(y, dmi, hbase)
    return _tc3(acc)


# speedup vs baseline: 3.9798x; 1.0227x over previous
import functools

import jax
import jax.numpy as jnp
from jax import lax
from jax.experimental import pallas as pl
from jax.experimental.pallas import tpu as pltpu
from jax.experimental.pallas import tpu_sc as plsc

N_ATOMS = 10000
N_PAIRS = 320000
N_EMB = 128
N_DIST = 100
N_HID = 64

NC = 2
NS = 16
NW = NC * NS

K = 2
_CP = N_PAIRS // K

_B1 = 1000


def _tc1_body(af_ref, wcf_ref, wfc_ref, bcf_ref, bdf_ref, afh_ref, hbase_ref):
    afh = jnp.dot(af_ref[...], wcf_ref[...],
                  preferred_element_type=jnp.float32) + bcf_ref[...]
    afh_ref[...] = jnp.concatenate([afh, jnp.zeros_like(afh)], axis=1)
    oii = jnp.tanh(jnp.dot(afh * bdf_ref[...], wfc_ref[...],
                           preferred_element_type=jnp.float32))
    hbase_ref[...] = (1.0 / (NC * K)) * (af_ref[...] - oii)


def _tc1(af, wcf, wfc, bcf, bdf):
    grid = (N_ATOMS // _B1,)
    return pl.pallas_call(
        _tc1_body,
        grid=grid,
        in_specs=[
            pl.BlockSpec((_B1, N_EMB), lambda i: (i, 0)),
            pl.BlockSpec((N_EMB, N_HID), lambda i: (0, 0)),
            pl.BlockSpec((N_HID, N_EMB), lambda i: (0, 0)),
            pl.BlockSpec((1, N_HID), lambda i: (0, 0)),
            pl.BlockSpec((1, N_HID), lambda i: (0, 0)),
        ],
        out_specs=[
            pl.BlockSpec((_B1, N_EMB), lambda i: (i, 0)),
            pl.BlockSpec((_B1, N_EMB), lambda i: (i, 0)),
        ],
        out_shape=[
            jax.ShapeDtypeStruct((N_ATOMS, N_EMB), jnp.float32),
            jax.ShapeDtypeStruct((N_ATOMS, N_EMB), jnp.float32),
        ],
    )(af, wcf, wfc, bcf, bdf)


_P_SC = _CP // NC
_SCH = 128
_P_T0 = 128 * ((_P_SC // 128 + NS - 1) // NS)
_N_T0 = _P_T0 // _SCH
_N_T15 = (_P_SC - (NS - 1) * _P_T0) // _SCH
assert _P_SC % 128 == 0 and _N_T15 > 0
_ROWS_T = 624
_ROWS_TAIL = N_ATOMS - NS * _ROWS_T


def _sc_gather_body(afh_hbm, idx_hbm, out_hbm, idx_v, rows_v, afh_sh,
                    sem_i, sem_g, sem_s):
    c = lax.axis_index("c")
    s = lax.axis_index("s")
    pltpu.sync_copy(afh_hbm.at[pl.ds(s * _ROWS_T, _ROWS_T)],
                    afh_sh.at[pl.ds(s * _ROWS_T, _ROWS_T)])

    @pl.when(s == 0)
    def _():
        pltpu.sync_copy(afh_hbm.at[pl.ds(NS * _ROWS_T, _ROWS_TAIL)],
                        afh_sh.at[pl.ds(NS * _ROWS_T, _ROWS_TAIL)])

    plsc.subcore_barrier()

    base = c * _P_SC + s * _P_T0
    n = jnp.where(s == NS - 1, _N_T15, _N_T0)

    def issue_idx(i, b):
        off = pl.multiple_of(base + i * _SCH, 8)
        pltpu.async_copy(idx_hbm.at[pl.ds(off, _SCH)], idx_v.at[b],
                         sem_i.at[b])

    def drain_i(b):
        pltpu.make_async_copy(idx_hbm.at[pl.ds(0, _SCH)], idx_v.at[b],
                              sem_i.at[b]).wait()

    def drain_rows(sem, b):
        pltpu.make_async_copy(out_hbm.at[pl.ds(0, _SCH)], rows_v.at[b],
                              sem.at[b]).wait()

    issue_idx(0, 0)

    def body(i, carry):
        b = i % 2
        nb = 1 - b

        @pl.when(i + 1 < n)
        def _():
            issue_idx(i + 1, nb)

        @pl.when(i >= 2)
        def _():
            drain_rows(sem_s, b)

        drain_i(b)
        pltpu.async_copy(afh_sh.at[idx_v.at[b]], rows_v.at[b], sem_g.at[b])
        drain_rows(sem_g, b)
        off = pl.multiple_of(base + i * _SCH, 8)
        pltpu.async_copy(rows_v.at[b], out_hbm.at[pl.ds(off, _SCH)],
                         sem_s.at[b])
        return carry

    lax.fori_loop(0, n, body, 0)
    drain_rows(sem_s, 0)
    drain_rows(sem_s, 1)


def _sc_gather(afh, dmj):
    mesh = plsc.VectorSubcoreMesh(core_axis_name="c", subcore_axis_name="s",
                                  num_cores=NC, num_subcores=NS)
    call = pl.kernel(
        _sc_gather_body,
        out_type=jax.ShapeDtypeStruct((_CP, N_EMB), jnp.float32),
        mesh=mesh,
        scratch_types=[
            pltpu.VMEM((2, _SCH), jnp.int32),
            pltpu.VMEM((2, _SCH, N_EMB), jnp.float32),
            pltpu.VMEM_SHARED((N_ATOMS, N_EMB), jnp.float32),
            pltpu.SemaphoreType.DMA((2,)),
            pltpu.SemaphoreType.DMA((2,)),
            pltpu.SemaphoreType.DMA((2,)),
        ],
    )
    return call(afh, dmj)


_B2 = 3200


def _tc2_body(d_ref, g_ref, wdf_ref, wfc_ref, bdf_ref, y_ref):
    dh = jnp.dot(d_ref[...], wdf_ref[...],
                 preferred_element_type=jnp.float32) + bdf_ref[...]
    g = g_ref[:, :N_HID]
    y_ref[...] = jnp.tanh(jnp.dot(dh * g, wfc_ref[...],
                                  preferred_element_type=jnp.float32))


def _tc2(k, dist, gathered, wdf, wfc, bdf):
    grid = (_CP // _B2,)
    k_off = k * (_CP // _B2)
    return pl.pallas_call(
        _tc2_body,
        grid=grid,
        in_specs=[
            pl.BlockSpec((_B2, N_DIST), lambda i: (k_off + i, 0)),
            pl.BlockSpec((_B2, N_EMB), lambda i: (i, 0)),
            pl.BlockSpec((N_DIST, N_HID), lambda i: (0, 0)),
            pl.BlockSpec((N_HID, N_EMB), lambda i: (0, 0)),
            pl.BlockSpec((1, N_HID), lambda i: (0, 0)),
        ],
        out_specs=pl.BlockSpec((_B2, N_EMB), lambda i: (i, 0)),
        out_shape=jax.ShapeDtypeStruct((_CP, N_EMB), jnp.float32),
    )(dist, gathered, wdf, wfc, bdf)


def _sc_scatter_body(y_hbm, dmi_hbm, hbase_hbm, out_hbm, idx_v, rows_v,
                     acc_sh, sem_i, sem_g, sem_a):
    c = lax.axis_index("c")
    s = lax.axis_index("s")
    pltpu.sync_copy(hbase_hbm.at[pl.ds(s * _ROWS_T, _ROWS_T)],
                    acc_sh.at[pl.ds(s * _ROWS_T, _ROWS_T)])

    @pl.when(s == 0)
    def _():
        pltpu.sync_copy(hbase_hbm.at[pl.ds(NS * _ROWS_T, _ROWS_TAIL)],
                        acc_sh.at[pl.ds(NS * _ROWS_T, _ROWS_TAIL)])

    plsc.subcore_barrier()

    base = c * _P_SC + s * _P_T0
    n = jnp.where(s == NS - 1, _N_T15, _N_T0)

    def issue_load(i, b):
        off = pl.multiple_of(base + i * _SCH, 8)
        pltpu.async_copy(dmi_hbm.at[pl.ds(off, _SCH)], idx_v.at[b],
                         sem_i.at[b])
        pltpu.async_copy(y_hbm.at[pl.ds(off, _SCH)], rows_v.at[b],
                         sem_g.at[b])

    def drain_i(b):
        pltpu.make_async_copy(dmi_hbm.at[pl.ds(0, _SCH)], idx_v.at[b],
                              sem_i.at[b]).wait()

    def drain_rows(sem, b):
        pltpu.make_async_copy(y_hbm.at[pl.ds(0, _SCH)], rows_v.at[b],
                              sem.at[b]).wait()

    issue_load(0, 0)

    def body(i, carry):
        b = i % 2
        nb = 1 - b

        @pl.when(i + 1 < n)
        def _():
            @pl.when(i >= 1)
            def _():
                drain_rows(sem_a, nb)
            issue_load(i + 1, nb)

        drain_i(b)
        drain_rows(sem_g, b)
        pltpu.async_copy(rows_v.at[b], acc_sh.at[idx_v.at[b]], sem_a.at[b],
                         add=True)
        return carry

    lax.fori_loop(0, n, body, 0)
    drain_rows(sem_a, 0)
    drain_rows(sem_a, 1)
    plsc.subcore_barrier()
    pltpu.sync_copy(acc_sh.at[pl.ds(s * _ROWS_T, _ROWS_T)],
                    out_hbm.at[c, pl.ds(s * _ROWS_T, _ROWS_T)])

    @pl.when(s == 0)
    def _():
        pltpu.sync_copy(acc_sh.at[pl.ds(NS * _ROWS_T, _ROWS_TAIL)],
                        out_hbm.at[c, pl.ds(NS * _ROWS_T, _ROWS_TAIL)])


def _sc_scatter(y, dmi, hbase):
    mesh = plsc.VectorSubcoreMesh(core_axis_name="c", subcore_axis_name="s",
                                  num_cores=NC, num_subcores=NS)
    call = pl.kernel(
        _sc_scatter_body,
        out_type=jax.ShapeDtypeStruct((NC, N_ATOMS, N_EMB), jnp.float32),
        mesh=mesh,
        scratch_types=[
            pltpu.VMEM((2, _SCH), jnp.int32),
            pltpu.VMEM((2, _SCH, N_EMB), jnp.float32),
            pltpu.VMEM_SHARED((N_ATOMS, N_EMB), jnp.float32),
            pltpu.SemaphoreType.DMA((2,)),
            pltpu.SemaphoreType.DMA((2,)),
            pltpu.SemaphoreType.DMA((2,)),
        ],
    )
    return call(y, dmi, hbase)


def _tc3_body(*refs):
    acc_refs, out_ref = refs[:-1], refs[-1]
    total = acc_refs[0][0] + acc_refs[0][1]
    for a in acc_refs[1:]:
        total = total + a[0] + a[1]
    out_ref[...] = total


def _tc3(accs):
    grid = (N_ATOMS // _B1,)
    return pl.pallas_call(
        _tc3_body,
        grid=grid,
        in_specs=[pl.BlockSpec((NC, _B1, N_EMB), lambda i: (0, i, 0))
                  for _ in accs],
        out_specs=pl.BlockSpec((_B1, N_EMB), lambda i: (i, 0)),
        out_shape=jax.ShapeDtypeStruct((N_ATOMS, N_EMB), jnp.float32),
    )(*accs)


def kernel(atom_features, distance, atom_membership, distance_membership_i,
           distance_membership_j, W_cf, W_df, W_fc, b_cf, b_df):
    del atom_membership
    dmi = distance_membership_i.astype(jnp.int32)
    dmj = distance_membership_j.astype(jnp.int32)
    bcf2 = b_cf.reshape(1, N_HID)
    bdf2 = b_df.reshape(1, N_HID)

    afh, hbase = _tc1(atom_features, W_cf, W_fc, bcf2, bdf2)
    accs = []
    for k in range(K):
        sl = slice(k * _CP, (k + 1) * _CP)
        g_k = _sc_gather(afh, dmj[sl])
        y_k = _tc2(k, distance, g_k, W_df, W_fc, bdf2)
        accs.append(_sc_scatter(y_k, dmi[sl], hbase))
    return _tc3(accs)


# speedup vs baseline: 5.1873x; 1.3034x over previous
import functools

import jax
import jax.numpy as jnp
from jax import lax
from jax.experimental import pallas as pl
from jax.experimental.pallas import tpu as pltpu
from jax.experimental.pallas import tpu_sc as plsc

N_ATOMS = 10000
N_PAIRS = 320000
N_EMB = 128
N_DIST = 100
N_HID = 64

NC = 2
NS = 16
NW = NC * NS

K = 2
_CP = N_PAIRS // K

_B1 = 1000


def _tc1_body(af_ref, wcf_ref, wfc_ref, bcf_ref, bdf_ref, afh_ref, hbase_ref):
    afh = jnp.dot(af_ref[...], wcf_ref[...],
                  preferred_element_type=jnp.float32) + bcf_ref[...]
    afh_ref[...] = jnp.concatenate([afh, jnp.zeros_like(afh)], axis=1)
    oii = jnp.tanh(jnp.dot(afh * bdf_ref[...], wfc_ref[...],
                           preferred_element_type=jnp.float32))
    hbase_ref[...] = (1.0 / (NC * K)) * (af_ref[...] - oii)


def _tc1(af, wcf, wfc, bcf, bdf):
    grid = (N_ATOMS // _B1,)
    return pl.pallas_call(
        _tc1_body,
        grid=grid,
        in_specs=[
            pl.BlockSpec((_B1, N_EMB), lambda i: (i, 0)),
            pl.BlockSpec((N_EMB, N_HID), lambda i: (0, 0)),
            pl.BlockSpec((N_HID, N_EMB), lambda i: (0, 0)),
            pl.BlockSpec((1, N_HID), lambda i: (0, 0)),
            pl.BlockSpec((1, N_HID), lambda i: (0, 0)),
        ],
        out_specs=[
            pl.BlockSpec((_B1, N_EMB), lambda i: (i, 0)),
            pl.BlockSpec((_B1, N_EMB), lambda i: (i, 0)),
        ],
        out_shape=[
            jax.ShapeDtypeStruct((N_ATOMS, N_EMB), jnp.float32),
            jax.ShapeDtypeStruct((N_ATOMS, N_EMB), jnp.float32),
        ],
    )(af, wcf, wfc, bcf, bdf)


_P_SC = _CP // NC
_SCH = 128
_P_T0 = 128 * ((_P_SC // 128 + NS - 1) // NS)
_N_T0 = _P_T0 // _SCH
_N_T15 = (_P_SC - (NS - 1) * _P_T0) // _SCH
assert _P_SC % 128 == 0 and _N_T15 > 0
_ROWS_T = 624
_ROWS_TAIL = N_ATOMS - NS * _ROWS_T


def _sc_gather_body(afh_hbm, idx_hbm, out_hbm, idx_v, rows_v, afh_sh,
                    sem_i, sem_g, sem_s):
    c = lax.axis_index("c")
    s = lax.axis_index("s")
    pltpu.sync_copy(afh_hbm.at[pl.ds(s * _ROWS_T, _ROWS_T)],
                    afh_sh.at[pl.ds(s * _ROWS_T, _ROWS_T)])

    @pl.when(s == 0)
    def _():
        pltpu.sync_copy(afh_hbm.at[pl.ds(NS * _ROWS_T, _ROWS_TAIL)],
                        afh_sh.at[pl.ds(NS * _ROWS_T, _ROWS_TAIL)])

    plsc.subcore_barrier()

    base = c * _P_SC + s * _P_T0
    n = jnp.where(s == NS - 1, _N_T15, _N_T0)

    def issue_idx(i, b):
        off = pl.multiple_of(base + i * _SCH, 8)
        pltpu.async_copy(idx_hbm.at[pl.ds(off, _SCH)], idx_v.at[b],
                         sem_i.at[b])

    def drain_i(b):
        pltpu.make_async_copy(idx_hbm.at[pl.ds(0, _SCH)], idx_v.at[b],
                              sem_i.at[b]).wait()

    def drain_rows(sem, b):
        pltpu.make_async_copy(out_hbm.at[pl.ds(0, _SCH)], rows_v.at[b],
                              sem.at[b]).wait()

    issue_idx(0, 0)

    def body(i, carry):
        b = i % 2
        nb = 1 - b

        @pl.when(i + 1 < n)
        def _():
            issue_idx(i + 1, nb)

        @pl.when(i >= 2)
        def _():
            drain_rows(sem_s, b)

        drain_i(b)
        pltpu.async_copy(afh_sh.at[idx_v.at[b]], rows_v.at[b], sem_g.at[b])
        drain_rows(sem_g, b)
        off = pl.multiple_of(base + i * _SCH, 8)
        pltpu.async_copy(rows_v.at[b], out_hbm.at[pl.ds(off, _SCH)],
                         sem_s.at[b])
        return carry

    lax.fori_loop(0, n, body, 0)
    drain_rows(sem_s, 0)
    drain_rows(sem_s, 1)


def _sc_gather(afh, dmj):
    mesh = plsc.VectorSubcoreMesh(core_axis_name="c", subcore_axis_name="s",
                                  num_cores=NC, num_subcores=NS)
    call = pl.kernel(
        _sc_gather_body,
        out_type=jax.ShapeDtypeStruct((_CP, N_EMB), jnp.float32),
        mesh=mesh,
        scratch_types=[
            pltpu.VMEM((2, _SCH), jnp.int32),
            pltpu.VMEM((2, _SCH, N_EMB), jnp.float32),
            pltpu.VMEM_SHARED((N_ATOMS, N_EMB), jnp.float32),
            pltpu.SemaphoreType.DMA((2,)),
            pltpu.SemaphoreType.DMA((2,)),
            pltpu.SemaphoreType.DMA((2,)),
        ],
    )
    return call(afh, dmj)


_B2 = 3200


def _tc2_body(dt_ref, g_ref, wdf_ref, wfc_ref, bdf_ref, y_ref):
    dh = lax.dot_general(dt_ref[...], wdf_ref[...],
                         dimension_numbers=(((0,), (0,)), ((), ())),
                         preferred_element_type=jnp.float32) + bdf_ref[...]
    g = g_ref[:, :N_HID]
    y_ref[...] = jnp.tanh(jnp.dot(dh * g, wfc_ref[...],
                                  preferred_element_type=jnp.float32))


def _tc2(k, dist_t, gathered, wdf, wfc, bdf):
    grid = (_CP // _B2,)
    k_off = k * (_CP // _B2)
    return pl.pallas_call(
        _tc2_body,
        grid=grid,
        in_specs=[
            pl.BlockSpec((N_DIST, _B2), lambda i: (0, k_off + i)),
            pl.BlockSpec((_B2, N_EMB), lambda i: (i, 0)),
            pl.BlockSpec((N_DIST, N_HID), lambda i: (0, 0)),
            pl.BlockSpec((N_HID, N_EMB), lambda i: (0, 0)),
            pl.BlockSpec((1, N_HID), lambda i: (0, 0)),
        ],
        out_specs=pl.BlockSpec((_B2, N_EMB), lambda i: (i, 0)),
        out_shape=jax.ShapeDtypeStruct((_CP, N_EMB), jnp.float32),
    )(dist_t, gathered, wdf, wfc, bdf)


def _sc_scatter_body(y_hbm, dmi_hbm, hbase_hbm, out_hbm, idx_v, rows_v,
                     acc_sh, sem_i, sem_g, sem_a):
    c = lax.axis_index("c")
    s = lax.axis_index("s")
    pltpu.sync_copy(hbase_hbm.at[pl.ds(s * _ROWS_T, _ROWS_T)],
                    acc_sh.at[pl.ds(s * _ROWS_T, _ROWS_T)])

    @pl.when(s == 0)
    def _():
        pltpu.sync_copy(hbase_hbm.at[pl.ds(NS * _ROWS_T, _ROWS_TAIL)],
                        acc_sh.at[pl.ds(NS * _ROWS_T, _ROWS_TAIL)])

    plsc.subcore_barrier()

    base = c * _P_SC + s * _P_T0
    n = jnp.where(s == NS - 1, _N_T15, _N_T0)

    def issue_load(i, b):
        off = pl.multiple_of(base + i * _SCH, 8)
        pltpu.async_copy(dmi_hbm.at[pl.ds(off, _SCH)], idx_v.at[b],
                         sem_i.at[b])
        pltpu.async_copy(y_hbm.at[pl.ds(off, _SCH)], rows_v.at[b],
                         sem_g.at[b])

    def drain_i(b):
        pltpu.make_async_copy(dmi_hbm.at[pl.ds(0, _SCH)], idx_v.at[b],
                              sem_i.at[b]).wait()

    def drain_rows(sem, b):
        pltpu.make_async_copy(y_hbm.at[pl.ds(0, _SCH)], rows_v.at[b],
                              sem.at[b]).wait()

    issue_load(0, 0)

    def body(i, carry):
        b = i % 2
        nb = 1 - b

        @pl.when(i + 1 < n)
        def _():
            @pl.when(i >= 1)
            def _():
                drain_rows(sem_a, nb)
            issue_load(i + 1, nb)

        drain_i(b)
        drain_rows(sem_g, b)
        pltpu.async_copy(rows_v.at[b], acc_sh.at[idx_v.at[b]], sem_a.at[b],
                         add=True)
        return carry

    lax.fori_loop(0, n, body, 0)
    drain_rows(sem_a, 0)
    drain_rows(sem_a, 1)
    plsc.subcore_barrier()
    pltpu.sync_copy(acc_sh.at[pl.ds(s * _ROWS_T, _ROWS_T)],
                    out_hbm.at[c, pl.ds(s * _ROWS_T, _ROWS_T)])

    @pl.when(s == 0)
    def _():
        pltpu.sync_copy(acc_sh.at[pl.ds(NS * _ROWS_T, _ROWS_TAIL)],
                        out_hbm.at[c, pl.ds(NS * _ROWS_T, _ROWS_TAIL)])


def _sc_scatter(y, dmi, hbase):
    mesh = plsc.VectorSubcoreMesh(core_axis_name="c", subcore_axis_name="s",
                                  num_cores=NC, num_subcores=NS)
    call = pl.kernel(
        _sc_scatter_body,
        out_type=jax.ShapeDtypeStruct((NC, N_ATOMS, N_EMB), jnp.float32),
        mesh=mesh,
        scratch_types=[
            pltpu.VMEM((2, _SCH), jnp.int32),
            pltpu.VMEM((2, _SCH, N_EMB), jnp.float32),
            pltpu.VMEM_SHARED((N_ATOMS, N_EMB), jnp.float32),
            pltpu.SemaphoreType.DMA((2,)),
            pltpu.SemaphoreType.DMA((2,)),
            pltpu.SemaphoreType.DMA((2,)),
        ],
    )
    return call(y, dmi, hbase)


def _tc3_body(*refs):
    acc_refs, out_ref = refs[:-1], refs[-1]
    total = acc_refs[0][0] + acc_refs[0][1]
    for a in acc_refs[1:]:
        total = total + a[0] + a[1]
    out_ref[...] = total


def _tc3(accs):
    grid = (N_ATOMS // _B1,)
    return pl.pallas_call(
        _tc3_body,
        grid=grid,
        in_specs=[pl.BlockSpec((NC, _B1, N_EMB), lambda i: (0, i, 0))
                  for _ in accs],
        out_specs=pl.BlockSpec((_B1, N_EMB), lambda i: (i, 0)),
        out_shape=jax.ShapeDtypeStruct((N_ATOMS, N_EMB), jnp.float32),
    )(*accs)


def kernel(atom_features, distance, atom_membership, distance_membership_i,
           distance_membership_j, W_cf, W_df, W_fc, b_cf, b_df):
    del atom_membership
    dmi = distance_membership_i.astype(jnp.int32)
    dmj = distance_membership_j.astype(jnp.int32)
    bcf2 = b_cf.reshape(1, N_HID)
    bdf2 = b_df.reshape(1, N_HID)

    dist_t = distance.T
    afh, hbase = _tc1(atom_features, W_cf, W_fc, bcf2, bdf2)
    accs = []
    for k in range(K):
        sl = slice(k * _CP, (k + 1) * _CP)
        g_k = _sc_gather(afh, dmj[sl])
        y_k = _tc2(k, dist_t, g_k, W_df, W_fc, bdf2)
        accs.append(_sc_scatter(y_k, dmi[sl], hbase))
    return _tc3(accs)


# speedup vs baseline: 5.3324x; 1.0280x over previous
import functools

import jax
import jax.numpy as jnp
from jax import lax
from jax.experimental import pallas as pl
from jax.experimental.pallas import tpu as pltpu
from jax.experimental.pallas import tpu_sc as plsc

N_ATOMS = 10000
N_PAIRS = 320000
N_EMB = 128
N_DIST = 100
N_HID = 64

NC = 2
NS = 16
NW = NC * NS

K = 2
_CP = N_PAIRS // K

_B1 = 1000


def _tc1_body(af_ref, wcf_ref, wfc_ref, bcf_ref, bdf_ref, afh_ref, hbase_ref):
    afh = jnp.dot(af_ref[...], wcf_ref[...],
                  preferred_element_type=jnp.float32) + bcf_ref[...]
    afh_ref[...] = jnp.concatenate([afh, jnp.zeros_like(afh)], axis=1)
    oii = jnp.tanh(jnp.dot(afh * bdf_ref[...], wfc_ref[...],
                           preferred_element_type=jnp.float32))
    hbase_ref[...] = (1.0 / (NC * K)) * (af_ref[...] - oii)


def _tc1(af, wcf, wfc, bcf, bdf):
    grid = (N_ATOMS // _B1,)
    return pl.pallas_call(
        _tc1_body,
        grid=grid,
        in_specs=[
            pl.BlockSpec((_B1, N_EMB), lambda i: (i, 0)),
            pl.BlockSpec((N_EMB, N_HID), lambda i: (0, 0)),
            pl.BlockSpec((N_HID, N_EMB), lambda i: (0, 0)),
            pl.BlockSpec((1, N_HID), lambda i: (0, 0)),
            pl.BlockSpec((1, N_HID), lambda i: (0, 0)),
        ],
        out_specs=[
            pl.BlockSpec((_B1, N_EMB), lambda i: (i, 0)),
            pl.BlockSpec((_B1, N_EMB), lambda i: (i, 0)),
        ],
        out_shape=[
            jax.ShapeDtypeStruct((N_ATOMS, N_EMB), jnp.float32),
            jax.ShapeDtypeStruct((N_ATOMS, N_EMB), jnp.float32),
        ],
    )(af, wcf, wfc, bcf, bdf)


_P_SC = _CP // NC
_SCH = 128
_P_T0 = 128 * ((_P_SC // 128 + NS - 1) // NS)
_N_T0 = _P_T0 // _SCH
_N_T15 = (_P_SC - (NS - 1) * _P_T0) // _SCH
assert _P_SC % 128 == 0 and _N_T15 > 0
_ROWS_T = 624
_ROWS_TAIL = N_ATOMS - NS * _ROWS_T


def _sc_gather_body(afh_hbm, idx_hbm, out_hbm, idx_v, rows_v, afh_sh,
                    sem_i, sem_g, sem_s):
    c = lax.axis_index("c")
    s = lax.axis_index("s")
    pltpu.sync_copy(afh_hbm.at[pl.ds(s * _ROWS_T, _ROWS_T)],
                    afh_sh.at[pl.ds(s * _ROWS_T, _ROWS_T)])

    @pl.when(s == 0)
    def _():
        pltpu.sync_copy(afh_hbm.at[pl.ds(NS * _ROWS_T, _ROWS_TAIL)],
                        afh_sh.at[pl.ds(NS * _ROWS_T, _ROWS_TAIL)])

    plsc.subcore_barrier()

    base = c * _P_SC + s * _P_T0
    n = jnp.where(s == NS - 1, _N_T15, _N_T0)

    def issue_idx(i, b):
        off = pl.multiple_of(base + i * _SCH, 8)
        pltpu.async_copy(idx_hbm.at[pl.ds(off, _SCH)], idx_v.at[b],
                         sem_i.at[b])

    def drain_i(b):
        pltpu.make_async_copy(idx_hbm.at[pl.ds(0, _SCH)], idx_v.at[b],
                              sem_i.at[b]).wait()

    def drain_rows(sem, b):
        pltpu.make_async_copy(out_hbm.at[pl.ds(0, _SCH)], rows_v.at[b],
                              sem.at[b]).wait()

    issue_idx(0, 0)

    def body(i, carry):
        b = i % 2
        nb = 1 - b

        @pl.when(i + 1 < n)
        def _():
            issue_idx(i + 1, nb)

        @pl.when(i >= 2)
        def _():
            drain_rows(sem_s, b)

        drain_i(b)
        pltpu.async_copy(afh_sh.at[idx_v.at[b]], rows_v.at[b], sem_g.at[b])
        drain_rows(sem_g, b)
        off = pl.multiple_of(base + i * _SCH, 8)
        pltpu.async_copy(rows_v.at[b], out_hbm.at[pl.ds(off, _SCH)],
                         sem_s.at[b])
        return carry

    lax.fori_loop(0, n, body, 0)
    drain_rows(sem_s, 0)
    drain_rows(sem_s, 1)


def _sc_gather(afh, dmj):
    mesh = plsc.VectorSubcoreMesh(core_axis_name="c", subcore_axis_name="s",
                                  num_cores=NC, num_subcores=NS)
    call = pl.kernel(
        _sc_gather_body,
        out_type=jax.ShapeDtypeStruct((_CP, N_EMB), jnp.float32),
        mesh=mesh,
        scratch_types=[
            pltpu.VMEM((2, _SCH), jnp.int32),
            pltpu.VMEM((2, _SCH, N_EMB), jnp.float32),
            pltpu.VMEM_SHARED((N_ATOMS, N_EMB), jnp.float32),
            pltpu.SemaphoreType.DMA((2,)),
            pltpu.SemaphoreType.DMA((2,)),
            pltpu.SemaphoreType.DMA((2,)),
        ],
    )
    return call(afh, dmj)


_B2 = 6400


def _tc2_body(dt_ref, g_ref, wdf_ref, wfc_ref, bdf_ref, y_ref):
    dh = lax.dot_general(dt_ref[...], wdf_ref[...],
                         dimension_numbers=(((0,), (0,)), ((), ())),
                         preferred_element_type=jnp.float32) + bdf_ref[...]
    g = g_ref[:, :N_HID]
    y_ref[...] = jnp.tanh(jnp.dot(dh * g, wfc_ref[...],
                                  preferred_element_type=jnp.float32))


def _tc2(k, dist_t, gathered, wdf, wfc, bdf):
    grid = (_CP // _B2,)
    k_off = k * (_CP // _B2)
    return pl.pallas_call(
        _tc2_body,
        grid=grid,
        in_specs=[
            pl.BlockSpec((N_DIST, _B2), lambda i: (0, k_off + i)),
            pl.BlockSpec((_B2, N_EMB), lambda i: (i, 0)),
            pl.BlockSpec((N_DIST, N_HID), lambda i: (0, 0)),
            pl.BlockSpec((N_HID, N_EMB), lambda i: (0, 0)),
            pl.BlockSpec((1, N_HID), lambda i: (0, 0)),
        ],
        out_specs=pl.BlockSpec((_B2, N_EMB), lambda i: (i, 0)),
        out_shape=jax.ShapeDtypeStruct((_CP, N_EMB), jnp.float32),
    )(dist_t, gathered, wdf, wfc, bdf)


def _sc_scatter_body(y_hbm, dmi_hbm, hbase_hbm, out_hbm, idx_v, rows_v,
                     acc_sh, sem_i, sem_g, sem_a):
    c = lax.axis_index("c")
    s = lax.axis_index("s")
    pltpu.sync_copy(hbase_hbm.at[pl.ds(s * _ROWS_T, _ROWS_T)],
                    acc_sh.at[pl.ds(s * _ROWS_T, _ROWS_T)])

    @pl.when(s == 0)
    def _():
        pltpu.sync_copy(hbase_hbm.at[pl.ds(NS * _ROWS_T, _ROWS_TAIL)],
                        acc_sh.at[pl.ds(NS * _ROWS_T, _ROWS_TAIL)])

    plsc.subcore_barrier()

    base = c * _P_SC + s * _P_T0
    n = jnp.where(s == NS - 1, _N_T15, _N_T0)

    def issue_load(i, b):
        off = pl.multiple_of(base + i * _SCH, 8)
        pltpu.async_copy(dmi_hbm.at[pl.ds(off, _SCH)], idx_v.at[b],
                         sem_i.at[b])
        pltpu.async_copy(y_hbm.at[pl.ds(off, _SCH)], rows_v.at[b],
                         sem_g.at[b])

    def drain_i(b):
        pltpu.make_async_copy(dmi_hbm.at[pl.ds(0, _SCH)], idx_v.at[b],
                              sem_i.at[b]).wait()

    def drain_rows(sem, b):
        pltpu.make_async_copy(y_hbm.at[pl.ds(0, _SCH)], rows_v.at[b],
                              sem.at[b]).wait()

    issue_load(0, 0)

    def body(i, carry):
        b = i % 2
        nb = 1 - b

        @pl.when(i + 1 < n)
        def _():
            @pl.when(i >= 1)
            def _():
                drain_rows(sem_a, nb)
            issue_load(i + 1, nb)

        drain_i(b)
        drain_rows(sem_g, b)
        pltpu.async_copy(rows_v.at[b], acc_sh.at[idx_v.at[b]], sem_a.at[b],
                         add=True)
        return carry

    lax.fori_loop(0, n, body, 0)
    drain_rows(sem_a, 0)
    drain_rows(sem_a, 1)
    plsc.subcore_barrier()
    pltpu.sync_copy(acc_sh.at[pl.ds(s * _ROWS_T, _ROWS_T)],
                    out_hbm.at[c, pl.ds(s * _ROWS_T, _ROWS_T)])

    @pl.when(s == 0)
    def _():
        pltpu.sync_copy(acc_sh.at[pl.ds(NS * _ROWS_T, _ROWS_TAIL)],
                        out_hbm.at[c, pl.ds(NS * _ROWS_T, _ROWS_TAIL)])


def _sc_scatter(y, dmi, hbase):
    mesh = plsc.VectorSubcoreMesh(core_axis_name="c", subcore_axis_name="s",
                                  num_cores=NC, num_subcores=NS)
    call = pl.kernel(
        _sc_scatter_body,
        out_type=jax.ShapeDtypeStruct((NC, N_ATOMS, N_EMB), jnp.float32),
        mesh=mesh,
        scratch_types=[
            pltpu.VMEM((2, _SCH), jnp.int32),
            pltpu.VMEM((2, _SCH, N_EMB), jnp.float32),
            pltpu.VMEM_SHARED((N_ATOMS, N_EMB), jnp.float32),
            pltpu.SemaphoreType.DMA((2,)),
            pltpu.SemaphoreType.DMA((2,)),
            pltpu.SemaphoreType.DMA((2,)),
        ],
    )
    return call(y, dmi, hbase)


def _tc3_body(*refs):
    acc_refs, out_ref = refs[:-1], refs[-1]
    total = acc_refs[0][0] + acc_refs[0][1]
    for a in acc_refs[1:]:
        total = total + a[0] + a[1]
    out_ref[...] = total


def _tc3(accs):
    grid = (N_ATOMS // _B1,)
    return pl.pallas_call(
        _tc3_body,
        grid=grid,
        in_specs=[pl.BlockSpec((NC, _B1, N_EMB), lambda i: (0, i, 0))
                  for _ in accs],
        out_specs=pl.BlockSpec((_B1, N_EMB), lambda i: (i, 0)),
        out_shape=jax.ShapeDtypeStruct((N_ATOMS, N_EMB), jnp.float32),
    )(*accs)


def kernel(atom_features, distance, atom_membership, distance_membership_i,
           distance_membership_j, W_cf, W_df, W_fc, b_cf, b_df):
    del atom_membership
    dmi = distance_membership_i.astype(jnp.int32)
    dmj = distance_membership_j.astype(jnp.int32)
    bcf2 = b_cf.reshape(1, N_HID)
    bdf2 = b_df.reshape(1, N_HID)

    dist_t = distance.T
    afh, hbase = _tc1(atom_features, W_cf, W_fc, bcf2, bdf2)
    accs = []
    for k in range(K):
        sl = slice(k * _CP, (k + 1) * _CP)
        g_k = _sc_gather(afh, dmj[sl])
        y_k = _tc2(k, dist_t, g_k, W_df, W_fc, bdf2)
        accs.append(_sc_scatter(y_k, dmi[sl], hbase))
    return _tc3(accs)


# speedup vs baseline: 5.4764x; 1.0270x over previous
import functools

import jax
import jax.numpy as jnp
from jax import lax
from jax.experimental import pallas as pl
from jax.experimental.pallas import tpu as pltpu
from jax.experimental.pallas import tpu_sc as plsc

N_ATOMS = 10000
N_PAIRS = 320000
N_EMB = 128
N_DIST = 100
N_HID = 64

NC = 2
NS = 16
NW = NC * NS

K = 2
_CP = N_PAIRS // K

_B1 = 1000


def _tc1_body(af_ref, wcf_ref, wfc_ref, bcf_ref, bdf_ref, afh_ref, hbase_ref):
    afh = jnp.dot(af_ref[...], wcf_ref[...],
                  preferred_element_type=jnp.float32) + bcf_ref[...]
    afh_ref[...] = jnp.concatenate([afh, jnp.zeros_like(afh)], axis=1)
    oii = jnp.tanh(jnp.dot(afh * bdf_ref[...], wfc_ref[...],
                           preferred_element_type=jnp.float32))
    hbase_ref[...] = (1.0 / (NC * K)) * (af_ref[...] - oii)


def _tc1(af, wcf, wfc, bcf, bdf):
    grid = (N_ATOMS // _B1,)
    return pl.pallas_call(
        _tc1_body,
        grid=grid,
        in_specs=[
            pl.BlockSpec((_B1, N_EMB), lambda i: (i, 0)),
            pl.BlockSpec((N_EMB, N_HID), lambda i: (0, 0)),
            pl.BlockSpec((N_HID, N_EMB), lambda i: (0, 0)),
            pl.BlockSpec((1, N_HID), lambda i: (0, 0)),
            pl.BlockSpec((1, N_HID), lambda i: (0, 0)),
        ],
        out_specs=[
            pl.BlockSpec((_B1, N_EMB), lambda i: (i, 0)),
            pl.BlockSpec((_B1, N_EMB), lambda i: (i, 0)),
        ],
        out_shape=[
            jax.ShapeDtypeStruct((N_ATOMS, N_EMB), jnp.float32),
            jax.ShapeDtypeStruct((N_ATOMS, N_EMB), jnp.float32),
        ],
    )(af, wcf, wfc, bcf, bdf)


_P_SC = _CP // NC
_SCH = 128
_P_T0 = 128 * ((_P_SC // 128 + NS - 1) // NS)
_N_T0 = _P_T0 // _SCH
_N_T15 = (_P_SC - (NS - 1) * _P_T0) // _SCH
assert _P_SC % 128 == 0 and _N_T15 > 0
_ROWS_T = 624
_ROWS_TAIL = N_ATOMS - NS * _ROWS_T


def _sc_gather_body(afh_hbm, idx_hbm, out_hbm, idx_v, rows_v, afh_sh,
                    sem_i, sem_g, sem_s):
    c = lax.axis_index("c")
    s = lax.axis_index("s")
    pltpu.sync_copy(afh_hbm.at[pl.ds(s * _ROWS_T, _ROWS_T)],
                    afh_sh.at[pl.ds(s * _ROWS_T, _ROWS_T)])

    @pl.when(s == 0)
    def _():
        pltpu.sync_copy(afh_hbm.at[pl.ds(NS * _ROWS_T, _ROWS_TAIL)],
                        afh_sh.at[pl.ds(NS * _ROWS_T, _ROWS_TAIL)])

    plsc.subcore_barrier()

    base = c * _P_SC + s * _P_T0
    n = jnp.where(s == NS - 1, _N_T15, _N_T0)

    def issue_idx(i, b):
        off = pl.multiple_of(base + i * _SCH, 8)
        pltpu.async_copy(idx_hbm.at[pl.ds(off, _SCH)], idx_v.at[b],
                         sem_i.at[b])

    def drain_i(b):
        pltpu.make_async_copy(idx_hbm.at[pl.ds(0, _SCH)], idx_v.at[b],
                              sem_i.at[b]).wait()

    def drain_rows(sem, b):
        pltpu.make_async_copy(out_hbm.at[pl.ds(0, _SCH)], rows_v.at[b],
                              sem.at[b]).wait()

    issue_idx(0, 0)

    def body(i, carry):
        b = i % 2
        nb = 1 - b

        @pl.when(i + 1 < n)
        def _():
            issue_idx(i + 1, nb)

        @pl.when(i >= 2)
        def _():
            drain_rows(sem_s, b)

        drain_i(b)
        pltpu.async_copy(afh_sh.at[idx_v.at[b]], rows_v.at[b], sem_g.at[b])
        drain_rows(sem_g, b)
        off = pl.multiple_of(base + i * _SCH, 8)
        pltpu.async_copy(rows_v.at[b], out_hbm.at[pl.ds(off, _SCH)],
                         sem_s.at[b])
        return carry

    lax.fori_loop(0, n, body, 0)
    drain_rows(sem_s, 0)
    drain_rows(sem_s, 1)


def _sc_gather(afh, dmj):
    mesh = plsc.VectorSubcoreMesh(core_axis_name="c", subcore_axis_name="s",
                                  num_cores=NC, num_subcores=NS)
    call = pl.kernel(
        _sc_gather_body,
        out_type=jax.ShapeDtypeStruct((_CP, N_EMB), jnp.float32),
        mesh=mesh,
        scratch_types=[
            pltpu.VMEM((2, _SCH), jnp.int32),
            pltpu.VMEM((2, _SCH, N_EMB), jnp.float32),
            pltpu.VMEM_SHARED((N_ATOMS, N_EMB), jnp.float32),
            pltpu.SemaphoreType.DMA((2,)),
            pltpu.SemaphoreType.DMA((2,)),
            pltpu.SemaphoreType.DMA((2,)),
        ],
    )
    return call(afh, dmj)


_B2 = 12800


def _tc2_body(dt_ref, g_ref, wdf_ref, wfc_ref, bdf_ref, y_ref):
    dh = lax.dot_general(dt_ref[...], wdf_ref[...],
                         dimension_numbers=(((0,), (0,)), ((), ())),
                         preferred_element_type=jnp.float32) + bdf_ref[...]
    g = g_ref[:, :N_HID]
    y_ref[...] = jnp.tanh(jnp.dot(dh * g, wfc_ref[...],
                                  preferred_element_type=jnp.float32))


def _tc2(k, dist_t, gathered, wdf, wfc, bdf):
    grid = (_CP // _B2,)
    k_off = k * (_CP // _B2)
    return pl.pallas_call(
        _tc2_body,
        grid=grid,
        in_specs=[
            pl.BlockSpec((N_DIST, _B2), lambda i: (0, k_off + i)),
            pl.BlockSpec((_B2, N_EMB), lambda i: (i, 0)),
            pl.BlockSpec((N_DIST, N_HID), lambda i: (0, 0)),
            pl.BlockSpec((N_HID, N_EMB), lambda i: (0, 0)),
            pl.BlockSpec((1, N_HID), lambda i: (0, 0)),
        ],
        out_specs=pl.BlockSpec((_B2, N_EMB), lambda i: (i, 0)),
        out_shape=jax.ShapeDtypeStruct((_CP, N_EMB), jnp.float32),
    )(dist_t, gathered, wdf, wfc, bdf)


def _sc_scatter_body(y_hbm, dmi_hbm, hbase_hbm, out_hbm, idx_v, rows_v,
                     acc_sh, sem_i, sem_g, sem_a):
    c = lax.axis_index("c")
    s = lax.axis_index("s")
    pltpu.sync_copy(hbase_hbm.at[pl.ds(s * _ROWS_T, _ROWS_T)],
                    acc_sh.at[pl.ds(s * _ROWS_T, _ROWS_T)])

    @pl.when(s == 0)
    def _():
        pltpu.sync_copy(hbase_hbm.at[pl.ds(NS * _ROWS_T, _ROWS_TAIL)],
                        acc_sh.at[pl.ds(NS * _ROWS_T, _ROWS_TAIL)])

    plsc.subcore_barrier()

    base = c * _P_SC + s * _P_T0
    n = jnp.where(s == NS - 1, _N_T15, _N_T0)

    def issue_load(i, b):
        off = pl.multiple_of(base + i * _SCH, 8)
        pltpu.async_copy(dmi_hbm.at[pl.ds(off, _SCH)], idx_v.at[b],
                         sem_i.at[b])
        pltpu.async_copy(y_hbm.at[pl.ds(off, _SCH)], rows_v.at[b],
                         sem_g.at[b])

    def drain_i(b):
        pltpu.make_async_copy(dmi_hbm.at[pl.ds(0, _SCH)], idx_v.at[b],
                              sem_i.at[b]).wait()

    def drain_rows(sem, b):
        pltpu.make_async_copy(y_hbm.at[pl.ds(0, _SCH)], rows_v.at[b],
                              sem.at[b]).wait()

    issue_load(0, 0)

    def body(i, carry):
        b = i % 2
        nb = 1 - b

        @pl.when(i + 1 < n)
        def _():
            @pl.when(i >= 1)
            def _():
                drain_rows(sem_a, nb)
            issue_load(i + 1, nb)

        drain_i(b)
        drain_rows(sem_g, b)
        pltpu.async_copy(rows_v.at[b], acc_sh.at[idx_v.at[b]], sem_a.at[b],
                         add=True)
        return carry

    lax.fori_loop(0, n, body, 0)
    drain_rows(sem_a, 0)
    drain_rows(sem_a, 1)
    plsc.subcore_barrier()
    pltpu.sync_copy(acc_sh.at[pl.ds(s * _ROWS_T, _ROWS_T)],
                    out_hbm.at[c, pl.ds(s * _ROWS_T, _ROWS_T)])

    @pl.when(s == 0)
    def _():
        pltpu.sync_copy(acc_sh.at[pl.ds(NS * _ROWS_T, _ROWS_TAIL)],
                        out_hbm.at[c, pl.ds(NS * _ROWS_T, _ROWS_TAIL)])


def _sc_scatter(y, dmi, hbase):
    mesh = plsc.VectorSubcoreMesh(core_axis_name="c", subcore_axis_name="s",
                                  num_cores=NC, num_subcores=NS)
    call = pl.kernel(
        _sc_scatter_body,
        out_type=jax.ShapeDtypeStruct((NC, N_ATOMS, N_EMB), jnp.float32),
        mesh=mesh,
        scratch_types=[
            pltpu.VMEM((2, _SCH), jnp.int32),
            pltpu.VMEM((2, _SCH, N_EMB), jnp.float32),
            pltpu.VMEM_SHARED((N_ATOMS, N_EMB), jnp.float32),
            pltpu.SemaphoreType.DMA((2,)),
            pltpu.SemaphoreType.DMA((2,)),
            pltpu.SemaphoreType.DMA((2,)),
        ],
    )
    return call(y, dmi, hbase)


def _tc3_body(*refs):
    acc_refs, out_ref = refs[:-1], refs[-1]
    total = acc_refs[0][0] + acc_refs[0][1]
    for a in acc_refs[1:]:
        total = total + a[0] + a[1]
    out_ref[...] = total


def _tc3(accs):
    grid = (N_ATOMS // _B1,)
    return pl.pallas_call(
        _tc3_body,
        grid=grid,
        in_specs=[pl.BlockSpec((NC, _B1, N_EMB), lambda i: (0, i, 0))
                  for _ in accs],
        out_specs=pl.BlockSpec((_B1, N_EMB), lambda i: (i, 0)),
        out_shape=jax.ShapeDtypeStruct((N_ATOMS, N_EMB), jnp.float32),
    )(*accs)


def kernel(atom_features, distance, atom_membership, distance_membership_i,
           distance_membership_j, W_cf, W_df, W_fc, b_cf, b_df):
    del atom_membership
    dmi = distance_membership_i.astype(jnp.int32)
    dmj = distance_membership_j.astype(jnp.int32)
    bcf2 = b_cf.reshape(1, N_HID)
    bdf2 = b_df.reshape(1, N_HID)

    dist_t = distance.T
    afh, hbase = _tc1(atom_features, W_cf, W_fc, bcf2, bdf2)
    accs = []
    for k in range(K):
        sl = slice(k * _CP, (k + 1) * _CP)
        g_k = _sc_gather(afh, dmj[sl])
        y_k = _tc2(k, dist_t, g_k, W_df, W_fc, bdf2)
        accs.append(_sc_scatter(y_k, dmi[sl], hbase))
    return _tc3(accs)
